# baseline (speedup 1.0000x reference)
import functools

import jax
import jax.numpy as jnp
from jax.experimental import pallas as pl
from jax.experimental.pallas import tpu as pltpu
from jax.experimental.pallas import tpu_sc as plsc

_K = 512
_D = 32
_BETA = 0.5
_KT = 64

_NC = 2
_NS = 16
_NW = _NC * _NS
_N_TOK = 4096
_BPW = _N_TOK // _NW


def _combine8(q):
    return ((q[0] + q[4]) + (q[2] + q[6])) + ((q[1] + q[5]) + (q[3] + q[7]))


def _dist_tree(ps):
    cs = [_combine8(ps[8 * c:8 * c + 8]) for c in range(len(ps) // 8)]
    acc = cs[0]
    for c in cs[1:]:
        acc = acc + c
    return acc


def _vq_body(x_ref, W_ref, idx_ref, loss_ref):
    nb = pl.num_programs(0)
    i = pl.program_id(0)
    dd, hh, ww = x_ref.shape[1:]
    nt = hh * ww
    xT = x_ref[0].reshape(dd, nt)
    run_min = jnp.full((1, nt), jnp.inf, dtype=jnp.float32)
    run_idx = jnp.zeros((1, nt), dtype=jnp.int32)
    for t in range(_K // _KT):
        Wt = W_ref[t * _KT:(t + 1) * _KT, :]
        ps = []
        for d in range(_D):
            diff = Wt[:, d:d + 1] - xT[d:d + 1, :]
            ps.append(diff * diff)
        dist = _dist_tree(ps)
        tmin = jnp.min(dist, axis=0, keepdims=True)
        kio = jax.lax.broadcasted_iota(jnp.int32, (_KT, nt), 0)
        tidx = jnp.min(jnp.where(dist == tmin, kio, _K),
                       axis=0, keepdims=True) + t * _KT
        better = tmin < run_min
        run_min = jnp.where(better, tmin, run_min)
        run_idx = jnp.where(better, tidx, run_idx)
    idx_ref[0, :] = run_idx[0]
    part = jnp.sum(run_min, axis=(0, 1), keepdims=True)

    @pl.when(i == 0)
    def _init():
        loss_ref[...] = jnp.zeros_like(loss_ref)

    loss_ref[...] += part

    @pl.when(i == nb - 1)
    def _fini():
        loss_ref[...] *= (1.0 + _BETA) / (nb * dd * nt)


_sc_mesh = plsc.VectorSubcoreMesh(core_axis_name="c", subcore_axis_name="s")


_DP = 128


@functools.partial(
    pl.kernel,
    mesh=_sc_mesh,
    out_type=jax.ShapeDtypeStruct((_N_TOK, _DP), jnp.float32),
    scratch_types=[
        pltpu.VMEM((_BPW,), jnp.int32),
        pltpu.VMEM((_BPW, _DP), jnp.float32),
        pltpu.SemaphoreType.DMA,
    ],
)
def _sc_gather(W_hbm, idx_hbm, out_hbm, idx_v, rows_v, sem):
    wid = jax.lax.axis_index("s") * _NC + jax.lax.axis_index("c")
    base = wid * _BPW
    pltpu.sync_copy(idx_hbm.at[pl.ds(base, _BPW)], idx_v)
    pltpu.async_copy(W_hbm.at[idx_v], rows_v, sem).wait()
    pltpu.sync_copy(rows_v, out_hbm.at[pl.ds(base, _BPW)])


def kernel(x, W):
    b, d, h, w = x.shape
    nt = h * w
    idx, loss = pl.pallas_call(
        _vq_body,
        grid=(b,),
        in_specs=[
            pl.BlockSpec((1, d, h, w), lambda i: (i, 0, 0, 0)),
            pl.BlockSpec((_K, _D), lambda i: (0, 0)),
        ],
        out_specs=[
            pl.BlockSpec((1, nt), lambda i: (0, i)),
            pl.BlockSpec((1, 1), lambda i: (0, 0)),
        ],
        out_shape=[
            jax.ShapeDtypeStruct((1, b * nt), jnp.int32),
            jax.ShapeDtypeStruct((1, 1), jnp.float32),
        ],
    )(x, W)
    latent_indices = idx.reshape(b * nt)
    W_pad = jnp.zeros((_K, _DP), W.dtype).at[:, :d].set(W)
    xq_rows = _sc_gather(W_pad, latent_indices)
    x_q_st = jnp.transpose(
        xq_rows.reshape(b, h, w, _DP)[..., :d], (0, 3, 1, 2))
    return (x_q_st, loss[0, 0], latent_indices)


# speedup vs baseline: 1.8657x; 1.8657x over previous
import jax
import jax.numpy as jnp
from jax.experimental import pallas as pl
from jax.experimental.pallas import tpu as pltpu

_K = 512
_D = 32
_BETA = 0.5
_NCAND = 4


def _row(a, r):
    return a[r:r + 1, :]


def _tree_rows(p):
    acc = None
    for c in range(_D // 8):
        q = [_row(p, 8 * c + j) for j in range(8)]
        t = ((q[0] + q[4]) + (q[2] + q[6])) + ((q[1] + q[5]) + (q[3] + q[7]))
        acc = t if acc is None else acc + t
    return acc


def _vq_body(x_ref, W_ref, xq_ref, idx_ref, loss_ref):
    nb = pl.num_programs(0)
    i = pl.program_id(0)
    dd, hh, ww = x_ref.shape[1:]
    nt = hh * ww
    xT = x_ref[0].reshape(dd, nt)
    W = W_ref[...]

    wn_half = 0.5 * jnp.sum(W * W, axis=1, keepdims=True)
    s = jax.lax.dot_general(
        W, xT, (((1,), (0,)), ((), ())),
        preferred_element_type=jnp.float32,
        precision=jax.lax.Precision.HIGHEST)
    v = wn_half - s

    kio = jax.lax.broadcasted_iota(jnp.int32, (_K, nt), 0)
    cands = []
    for j in range(_NCAND):
        mj = jnp.min(v, axis=0, keepdims=True)
        cj = jnp.min(jnp.where(v == mj, kio, _K), axis=0, keepdims=True)
        cands.append(cj)
        if j < _NCAND - 1:
            v = jnp.where(kio == cj, jnp.inf, v)

    best_d = None
    best_c = None
    wcs = []
    for cj in cands:
        oh = (kio == cj).astype(jnp.float32)
        Wc = jax.lax.dot_general(
            W, oh, (((0,), (0,)), ((), ())),
            preferred_element_type=jnp.float32,
            precision=jax.lax.Precision.HIGHEST)
        wcs.append(Wc)
        diff = Wc - xT
        dj = _tree_rows(diff * diff)
        if best_d is None:
            best_d, best_c = dj, cj
        else:
            better = (dj < best_d) | ((dj == best_d) & (cj < best_c))
            best_d = jnp.where(better, dj, best_d)
            best_c = jnp.where(better, cj, best_c)

    idx_ref[0, :] = best_c[0]

    xqT = wcs[0]
    for j in range(1, _NCAND):
        xqT = jnp.where(best_c == cands[j], wcs[j], xqT)
    xq_ref[0] = (xT + (xqT - xT)).reshape(dd, hh, ww)

    part = jnp.sum(best_d, axis=(0, 1), keepdims=True)

    @pl.when(i == 0)
    def _init():
        loss_ref[...] = jnp.zeros_like(loss_ref)

    loss_ref[...] += part

    @pl.when(i == nb - 1)
    def _fini():
        loss_ref[...] *= (1.0 + _BETA) / (nb * dd * nt)


def kernel(x, W):
    b, d, h, w = x.shape
    nt = h * w
    xq, idx, loss = pl.pallas_call(
        _vq_body,
        grid=(b,),
        in_specs=[
            pl.BlockSpec((1, d, h, w), lambda i: (i, 0, 0, 0)),
            pl.BlockSpec((_K, _D), lambda i: (0, 0)),
        ],
        out_specs=[
            pl.BlockSpec((1, d, h, w), lambda i: (i, 0, 0, 0)),
            pl.BlockSpec((1, nt), lambda i: (0, i)),
            pl.BlockSpec((1, 1), lambda i: (0, 0)),
        ],
        out_shape=[
            jax.ShapeDtypeStruct((b, d, h, w), jnp.float32),
            jax.ShapeDtypeStruct((1, b * nt), jnp.int32),
            jax.ShapeDtypeStruct((1, 1), jnp.float32),
        ],
    )(x, W)
    return (xq, loss[0, 0], idx.reshape(b * nt))


# speedup vs baseline: 2.3908x; 1.2814x over previous
import jax
import jax.numpy as jnp
from jax.experimental import pallas as pl
from jax.experimental.pallas import tpu as pltpu

_K = 512
_D = 32
_BETA = 0.5
_NCAND = 4


def _row(a, r):
    return a[r:r + 1, :]


def _tree_rows(p):
    acc = None
    for c in range(_D // 8):
        q = [_row(p, 8 * c + j) for j in range(8)]
        t = ((q[0] + q[4]) + (q[2] + q[6])) + ((q[1] + q[5]) + (q[3] + q[7]))
        acc = t if acc is None else acc + t
    return acc


def _vq_body(x_ref, W_ref, xq_ref, idx_ref, loss_ref):
    nb = pl.num_programs(0)
    i = pl.program_id(0)
    dd, hh, ww = x_ref.shape[1:]
    nt = hh * ww
    xT = x_ref[0].reshape(dd, nt)
    W = W_ref[...]
    w_hi = W.astype(jnp.bfloat16)
    r1 = W - w_hi.astype(jnp.float32)
    w_mid = r1.astype(jnp.bfloat16)
    w_lo = (r1 - w_mid.astype(jnp.float32)).astype(jnp.bfloat16)

    wn_half = 0.5 * jnp.sum(W * W, axis=1, keepdims=True)
    s = jax.lax.dot_general(
        W, xT, (((1,), (0,)), ((), ())),
        preferred_element_type=jnp.float32,
        precision=jax.lax.Precision.HIGHEST)
    v = wn_half - s

    kio = jax.lax.broadcasted_iota(jnp.int32, (_K, nt), 0)
    cands = []
    for j in range(_NCAND):
        mj = jnp.min(v, axis=0, keepdims=True)
        cj = jnp.min(jnp.where(v == mj, kio, _K), axis=0, keepdims=True)
        cands.append(cj)
        if j < _NCAND - 1:
            v = jnp.where(kio == cj, jnp.inf, v)

    best_d = None
    best_c = None
    wcs = []
    dn = (((0,), (0,)), ((), ()))
    kio16 = jax.lax.broadcasted_iota(jnp.int16, (_K, nt), 0)
    for cj in cands:
        oh = jnp.where(kio16 == cj.astype(jnp.int16),
                       jnp.bfloat16(1), jnp.bfloat16(0))
        Wc = (jax.lax.dot_general(w_hi, oh, dn,
                                  preferred_element_type=jnp.float32)
              + jax.lax.dot_general(w_mid, oh, dn,
                                    preferred_element_type=jnp.float32)
              + jax.lax.dot_general(w_lo, oh, dn,
                                    preferred_element_type=jnp.float32)
              )
        wcs.append(Wc)
        diff = Wc - xT
        dj = _tree_rows(diff * diff)
        if best_d is None:
            best_d, best_c = dj, cj
        else:
            better = (dj < best_d) | ((dj == best_d) & (cj < best_c))
            best_d = jnp.where(better, dj, best_d)
            best_c = jnp.where(better, cj, best_c)

    idx_ref[0, :] = best_c[0]

    xqT = wcs[0]
    for j in range(1, _NCAND):
        xqT = jnp.where(best_c == cands[j], wcs[j], xqT)
    xq_ref[0] = (xT + (xqT - xT)).reshape(dd, hh, ww)

    part = jnp.sum(best_d, axis=(0, 1), keepdims=True)

    @pl.when(i == 0)
    def _init():
        loss_ref[...] = jnp.zeros_like(loss_ref)

    loss_ref[...] += part

    @pl.when(i == nb - 1)
    def _fini():
        loss_ref[...] *= (1.0 + _BETA) / (nb * dd * nt)


def kernel(x, W):
    b, d, h, w = x.shape
    nt = h * w
    xq, idx, loss = pl.pallas_call(
        _vq_body,
        grid=(b,),
        in_specs=[
            pl.BlockSpec((1, d, h, w), lambda i: (i, 0, 0, 0)),
            pl.BlockSpec((_K, _D), lambda i: (0, 0)),
        ],
        out_specs=[
            pl.BlockSpec((1, d, h, w), lambda i: (i, 0, 0, 0)),
            pl.BlockSpec((1, nt), lambda i: (0, i)),
            pl.BlockSpec((1, 1), lambda i: (0, 0)),
        ],
        out_shape=[
            jax.ShapeDtypeStruct((b, d, h, w), jnp.float32),
            jax.ShapeDtypeStruct((1, b * nt), jnp.int32),
            jax.ShapeDtypeStruct((1, 1), jnp.float32),
        ],
    )(x, W)
    return (xq, loss[0, 0], idx.reshape(b * nt))


# speedup vs baseline: 3.2074x; 1.3416x over previous
import jax
import jax.numpy as jnp
from jax.experimental import pallas as pl
from jax.experimental.pallas import tpu as pltpu

_K = 512
_D = 32
_BETA = 0.5
_NCAND = 4


def _row(a, r):
    return a[r:r + 1, :]


def _tree_rows(p):
    acc = None
    for c in range(_D // 8):
        q = [_row(p, 8 * c + j) for j in range(8)]
        t = ((q[0] + q[4]) + (q[2] + q[6])) + ((q[1] + q[5]) + (q[3] + q[7]))
        acc = t if acc is None else acc + t
    return acc


def _vq_body(x_ref, W_ref, xq_ref, idx_ref, loss_ref):
    nb = pl.num_programs(0)
    i = pl.program_id(0)
    dd, hh, ww = x_ref.shape[1:]
    nt = hh * ww
    xT = x_ref[0].reshape(dd, nt)
    W = W_ref[...]
    w_hi = W.astype(jnp.bfloat16)
    r1 = W - w_hi.astype(jnp.float32)
    w_mid = r1.astype(jnp.bfloat16)
    w_lo = (r1 - w_mid.astype(jnp.float32)).astype(jnp.bfloat16)

    x_hi = xT.astype(jnp.bfloat16)
    x_lo = (xT - x_hi.astype(jnp.float32)).astype(jnp.bfloat16)
    wn_half = 0.5 * jnp.sum(W * W, axis=1, keepdims=True)
    dnk = (((1,), (0,)), ((), ()))
    s = (jax.lax.dot_general(w_hi, x_hi, dnk,
                             preferred_element_type=jnp.float32)
         + jax.lax.dot_general(w_hi, x_lo, dnk,
                               preferred_element_type=jnp.float32)
         + jax.lax.dot_general(w_mid, x_hi, dnk,
                               preferred_element_type=jnp.float32))
    v = wn_half - s

    kio = jax.lax.broadcasted_iota(jnp.int32, (_K, nt), 0)
    cands = []
    for j in range(_NCAND):
        mj = jnp.min(v, axis=0, keepdims=True)
        cj = jnp.min(jnp.where(v == mj, kio, _K), axis=0, keepdims=True)
        cands.append(cj)
        if j < _NCAND - 1:
            v = jnp.where(kio == cj, jnp.inf, v)

    best_d = None
    best_c = None
    wcs = []
    dn = (((0,), (0,)), ((), ()))
    kio16 = jax.lax.broadcasted_iota(jnp.int16, (_K, nt), 0)
    w3 = jnp.concatenate([w_hi, w_mid, w_lo], axis=1)
    for cj in cands:
        oh = jnp.where(kio16 == cj.astype(jnp.int16),
                       jnp.bfloat16(1), jnp.bfloat16(0))
        g = jax.lax.dot_general(w3, oh, dn,
                                preferred_element_type=jnp.float32)
        Wc = (g[:_D] + g[_D:2 * _D]) + g[2 * _D:]
        wcs.append(Wc)
        diff = Wc - xT
        dj = _tree_rows(diff * diff)
        if best_d is None:
            best_d, best_c = dj, cj
        else:
            better = (dj < best_d) | ((dj == best_d) & (cj < best_c))
            best_d = jnp.where(better, dj, best_d)
            best_c = jnp.where(better, cj, best_c)

    idx_ref[0, :] = best_c[0]

    xqT = wcs[0]
    for j in range(1, _NCAND):
        xqT = jnp.where(best_c == cands[j], wcs[j], xqT)
    xq_ref[0] = (xT + (xqT - xT)).reshape(dd, hh, ww)

    part = jnp.sum(best_d, axis=(0, 1), keepdims=True)

    @pl.when(i == 0)
    def _init():
        loss_ref[...] = jnp.zeros_like(loss_ref)

    loss_ref[...] += part

    @pl.when(i == nb - 1)
    def _fini():
        loss_ref[...] *= (1.0 + _BETA) / (nb * dd * nt)


def kernel(x, W):
    b, d, h, w = x.shape
    nt = h * w
    xq, idx, loss = pl.pallas_call(
        _vq_body,
        grid=(b,),
        in_specs=[
            pl.BlockSpec((1, d, h, w), lambda i: (i, 0, 0, 0)),
            pl.BlockSpec((_K, _D), lambda i: (0, 0)),
        ],
        out_specs=[
            pl.BlockSpec((1, d, h, w), lambda i: (i, 0, 0, 0)),
            pl.BlockSpec((1, nt), lambda i: (0, i)),
            pl.BlockSpec((1, 1), lambda i: (0, 0)),
        ],
        out_shape=[
            jax.ShapeDtypeStruct((b, d, h, w), jnp.float32),
            jax.ShapeDtypeStruct((1, b * nt), jnp.int32),
            jax.ShapeDtypeStruct((1, 1), jnp.float32),
        ],
    )(x, W)
    return (xq, loss[0, 0], idx.reshape(b * nt))


# speedup vs baseline: 3.2347x; 1.0085x over previous
import jax
import jax.numpy as jnp
from jax.experimental import pallas as pl
from jax.experimental.pallas import tpu as pltpu

_K = 512
_D = 32
_BETA = 0.5
_NCAND = 4


def _row(a, r):
    return a[r:r + 1, :]


def _tree_rows(p):
    acc = None
    for c in range(_D // 8):
        q = [_row(p, 8 * c + j) for j in range(8)]
        t = ((q[0] + q[4]) + (q[2] + q[6])) + ((q[1] + q[5]) + (q[3] + q[7]))
        acc = t if acc is None else acc + t
    return acc


_DNK = (((1,), (0,)), ((), ()))
_DN0 = (((0,), (0,)), ((), ()))


def _batch_vq(xT, w_hi, w_mid, w3, wn_half):
    nt = xT.shape[1]
    x_hi = xT.astype(jnp.bfloat16)
    x_lo = (xT - x_hi.astype(jnp.float32)).astype(jnp.bfloat16)
    s = (jax.lax.dot_general(w_hi, x_hi, _DNK,
                             preferred_element_type=jnp.float32)
         + jax.lax.dot_general(w_hi, x_lo, _DNK,
                               preferred_element_type=jnp.float32)
         + jax.lax.dot_general(w_mid, x_hi, _DNK,
                               preferred_element_type=jnp.float32))
    v = wn_half - s

    kio = jax.lax.broadcasted_iota(jnp.int32, (_K, nt), 0)
    cands = []
    for j in range(_NCAND):
        mj = jnp.min(v, axis=0, keepdims=True)
        cj = jnp.min(jnp.where(v == mj, kio, _K), axis=0, keepdims=True)
        cands.append(cj)
        if j < _NCAND - 1:
            v = jnp.where(kio == cj, jnp.inf, v)

    best_d = None
    best_c = None
    wcs = []
    kio16 = jax.lax.broadcasted_iota(jnp.int16, (_K, nt), 0)
    for cj in cands:
        oh = jnp.where(kio16 == cj.astype(jnp.int16),
                       jnp.bfloat16(1), jnp.bfloat16(0))
        g = jax.lax.dot_general(w3, oh, _DN0,
                                preferred_element_type=jnp.float32)
        Wc = (g[:_D] + g[_D:2 * _D]) + g[2 * _D:]
        wcs.append(Wc)
        diff = Wc - xT
        dj = _tree_rows(diff * diff)
        if best_d is None:
            best_d, best_c = dj, cj
        else:
            better = (dj < best_d) | ((dj == best_d) & (cj < best_c))
            best_d = jnp.where(better, dj, best_d)
            best_c = jnp.where(better, cj, best_c)

    xqT = wcs[0]
    for j in range(1, _NCAND):
        xqT = jnp.where(best_c == cands[j], wcs[j], xqT)
    xqT = xT + (xqT - xT)
    return best_c, best_d, xqT


def _vq_body(x_ref, W_ref, xq_ref, idx_ref, loss_ref):
    nb, dd, hh, ww = x_ref.shape
    nt = hh * ww
    W = W_ref[...]
    w_hi = W.astype(jnp.bfloat16)
    r1 = W - w_hi.astype(jnp.float32)
    w_mid = r1.astype(jnp.bfloat16)
    w_lo = (r1 - w_mid.astype(jnp.float32)).astype(jnp.bfloat16)
    w3 = jnp.concatenate([w_hi, w_mid, w_lo], axis=1)
    wn_half = 0.5 * jnp.sum(W * W, axis=1, keepdims=True)

    total = None
    for b in range(nb):
        xT = x_ref[b].reshape(dd, nt)
        best_c, best_d, xqT = _batch_vq(xT, w_hi, w_mid, w3, wn_half)
        idx_ref[0, b * nt:(b + 1) * nt] = best_c[0]
        xq_ref[b] = xqT.reshape(dd, hh, ww)
        part = jnp.sum(best_d, axis=(0, 1), keepdims=True)
        total = part if total is None else total + part
    loss_ref[...] = total * ((1.0 + _BETA) / (nb * dd * nt))


def kernel(x, W):
    b, d, h, w = x.shape
    nt = h * w
    xq, idx, loss = pl.pallas_call(
        _vq_body,
        out_shape=[
            jax.ShapeDtypeStruct((b, d, h, w), jnp.float32),
            jax.ShapeDtypeStruct((1, b * nt), jnp.int32),
            jax.ShapeDtypeStruct((1, 1), jnp.float32),
        ],
    )(x, W)
    return (xq, loss[0, 0], idx.reshape(b * nt))


# speedup vs baseline: 3.2378x; 1.0009x over previous
import jax
import jax.numpy as jnp
from jax.experimental import pallas as pl

_K = 512
_D = 32
_BETA = 0.5
_NCAND = 4


def _row(a, r):
    return a[r:r + 1, :]


def _tree_rows(p):
    acc = None
    for c in range(_D // 8):
        q = [_row(p, 8 * c + j) for j in range(8)]
        t = ((q[0] + q[4]) + (q[2] + q[6])) + ((q[1] + q[5]) + (q[3] + q[7]))
        acc = t if acc is None else acc + t
    return acc


_DNK = (((1,), (0,)), ((), ()))
_DN0 = (((0,), (0,)), ((), ()))


def _batch_vq(xT, w_hi, w_mid, w3, wn_half):
    nt = xT.shape[1]
    x_hi = xT.astype(jnp.bfloat16)
    x_lo = (xT - x_hi.astype(jnp.float32)).astype(jnp.bfloat16)
    s = (jax.lax.dot_general(w_hi, x_hi, _DNK,
                             preferred_element_type=jnp.float32)
         + jax.lax.dot_general(w_hi, x_lo, _DNK,
                               preferred_element_type=jnp.float32)
         + jax.lax.dot_general(w_mid, x_hi, _DNK,
                               preferred_element_type=jnp.float32))
    v = wn_half - s

    kio = jax.lax.broadcasted_iota(jnp.int32, (_K, nt), 0)
    cands = []
    for j in range(_NCAND):
        mj = jnp.min(v, axis=0, keepdims=True)
        cj = jnp.min(jnp.where(v == mj, kio, _K), axis=0, keepdims=True)
        cands.append(cj)
        if j < _NCAND - 1:
            v = jnp.where(kio == cj, jnp.inf, v)

    best_d = None
    best_c = None
    wcs = []
    kio16 = jax.lax.broadcasted_iota(jnp.int16, (_K, nt), 0)
    for cj in cands:
        oh = jnp.where(kio16 == cj.astype(jnp.int16),
                       jnp.bfloat16(1), jnp.bfloat16(0))
        g = jax.lax.dot_general(w3, oh, _DN0,
                                preferred_element_type=jnp.float32)
        Wc = (g[:_D] + g[_D:2 * _D]) + g[2 * _D:]
        wcs.append(Wc)
        diff = Wc - xT
        dj = _tree_rows(diff * diff)
        if best_d is None:
            best_d, best_c = dj, cj
        else:
            better = (dj < best_d) | ((dj == best_d) & (cj < best_c))
            best_d = jnp.where(better, dj, best_d)
            best_c = jnp.where(better, cj, best_c)

    xqT = wcs[0]
    for j in range(1, _NCAND):
        xqT = jnp.where(best_c == cands[j], wcs[j], xqT)
    xqT = xT + (xqT - xT)
    return best_c, best_d, xqT


def _vq_body(x_ref, W_ref, xq_ref, idx_ref, loss_ref):
    nb, dd, hh, ww = x_ref.shape
    nt = hh * ww
    W = W_ref[...]
    w_hi = W.astype(jnp.bfloat16)
    r1 = W - w_hi.astype(jnp.float32)
    w_mid = r1.astype(jnp.bfloat16)
    w_lo = (r1 - w_mid.astype(jnp.float32)).astype(jnp.bfloat16)
    w3 = jnp.concatenate([w_hi, w_mid, w_lo], axis=1)
    wn_half = 0.5 * jnp.sum(W * W, axis=1, keepdims=True)

    total = None
    for b in range(nb):
        xT = x_ref[b].reshape(dd, nt)
        best_c, best_d, xqT = _batch_vq(xT, w_hi, w_mid, w3, wn_half)
        idx_ref[0, b * nt:(b + 1) * nt] = best_c[0]
        xq_ref[b] = xqT.reshape(dd, hh, ww)
        part = jnp.sum(best_d, axis=(0, 1), keepdims=True)
        total = part if total is None else total + part
    loss_ref[...] = total * ((1.0 + _BETA) / (nb * dd * nt))


def kernel(x, W):
    b, d, h, w = x.shape
    nt = h * w
    xq, idx, loss = pl.pallas_call(
        _vq_body,
        out_shape=[
            jax.ShapeDtypeStruct((b, d, h, w), jnp.float32),
            jax.ShapeDtypeStruct((1, b * nt), jnp.int32),
            jax.ShapeDtypeStruct((1, 1), jnp.float32),
        ],
    )(x, W)
    return (xq, loss[0, 0], idx.reshape(b * nt))
